# Initial kernel scaffold; baseline (speedup 1.0000x reference)
import jax
import jax.numpy as jnp
from jax.experimental import pallas as pl


def kernel(points, hash_table, offset_table, sparsity_encoding, m0, m1):
    raise NotImplementedError("write your pallas kernel here")


# speedup vs baseline: 8.1104x; 8.1104x over previous
import functools

import jax
import jax.numpy as jnp
import numpy as np
from jax import lax
from jax.experimental import pallas as pl
from jax.experimental.pallas import tpu as pltpu
from jax.experimental.pallas import tpu_sc as plsc

C1 = 1178101

NC = 2
NS = 16
L = 16
NW = NC * NS
NPAD = 1 << 20
CH = 2048
R = NPAD // (NW * CH)
GW = 128


def _sc_hash_lookup(T, O, C, oscale):
    mesh = plsc.VectorSubcoreMesh(core_axis_name="c", subcore_axis_name="s")

    @functools.partial(
        pl.kernel,
        mesh=mesh,
        out_type=jax.ShapeDtypeStruct((NPAD * C,), jnp.float32),
        compiler_params=pltpu.CompilerParams(needs_layout_passes=False),
        scratch_types=[
            pltpu.VMEM((CH,), jnp.int32),
            pltpu.VMEM((CH,), jnp.int32),
            pltpu.VMEM((CH,), jnp.int32),
            pltpu.VMEM((CH,), jnp.int32),
            pltpu.VMEM((CH,), jnp.int32),
            pltpu.VMEM((CH,), jnp.int32),
            pltpu.VMEM((CH,), jnp.int32),
            pltpu.VMEM((CH,), jnp.int32),
            pltpu.VMEM((CH,), jnp.int32),
            pltpu.VMEM((CH,), jnp.int32),
            pltpu.VMEM((CH,), jnp.int32),
            pltpu.VMEM((CH + L,), jnp.int32),
            pltpu.VMEM((CH + L,), jnp.int32),
            pltpu.VMEM((L, L), jnp.float32),
            pltpu.VMEM((L,), jnp.float32),
            pltpu.VMEM((CH * 16,), jnp.float32),
            pltpu.VMEM((T,), jnp.float32),
            pltpu.VMEM((8, 16), jnp.float32),
            pltpu.SemaphoreType.DMA,
            pltpu.SemaphoreType.DMA,
            pltpu.SemaphoreType.DMA,
        ],
    )
    def kern(px_h, py_h, pz_h, tbl_h, offp_h, sp_h, ttab_h, mm_h, out_h,
             p0v, p1v, p2v, ohv, hxv, hyv, hzv, cbv, offv, idxv, stv,
             hitrv, hitdv, stagev, dumv, zerov, ttabv, mmv, sem, sem2, sem3):
        wid = lax.axis_index("s") * NC + lax.axis_index("c")
        pltpu.sync_copy(ttab_h, ttabv)
        pltpu.sync_copy(mm_h, mmv)
        m0x = mmv[0, :]
        m0y = mmv[1, :]
        m0z = mmv[2, :]
        m1x = mmv[3, :]
        m1y = mmv[4, :]
        m1z = mmv[5, :]
        iota = lax.iota(jnp.int32, L)
        zvec = jnp.zeros((L,), jnp.float32)

        def zinit(i, _):
            zerov[pl.ds(i * L, L)] = zvec
            return 0

        lax.fori_loop(0, CH * 16 // L, zinit, 0)

        def round_body(r, _):
            base = wid * (R * CH) + r * CH
            pltpu.sync_copy(px_h.at[pl.ds(base, CH)], p0v)
            pltpu.sync_copy(py_h.at[pl.ds(base, CH)], p1v)
            pltpu.sync_copy(pz_h.at[pl.ds(base, CH)], p2v)

            def loop_a(i, _):
                s = pl.ds(i * L, L)
                a0 = p0v[s]
                a1 = p1v[s]
                a2 = p2v[s]
                f0 = a0.astype(jnp.float32)
                f1 = a1.astype(jnp.float32)
                f2 = a2.astype(jnp.float32)
                oh0 = (f0 * m1x).astype(jnp.int32) & (O - 1)
                oh1 = (f1 * m1y).astype(jnp.int32) & (O - 1)
                oh2 = (f2 * m1z).astype(jnp.int32) & (O - 1)
                ohv[s] = (oh0 * O + oh1) * O + oh2
                hxv[s] = (f0 * m0x).astype(jnp.int32)
                hyv[s] = (f1 * m0y).astype(jnp.int32)
                hzv[s] = (f2 * m0z).astype(jnp.int32)
                t0 = plsc.load_gather(ttabv, [a0])
                t1 = plsc.load_gather(ttabv, [a1])
                t2 = plsc.load_gather(ttabv, [a2])
                hk = (t0 + t1) + t2
                x = 256.0 * hk
                x = jnp.maximum(x, 0.0)
                x = jnp.minimum(x, 255.0)
                cbv[s] = x.astype(jnp.int32)
                return 0

            lax.fori_loop(0, CH // L, loop_a, 0)

            cps = [
                pltpu.async_copy(
                    offp_h.at[ohv.at[pl.ds(w * GW, GW)]],
                    offv.at[pl.ds(w * GW, GW)], sem)
                for w in range(CH // GW)
            ]
            for cp in cps:
                cp.wait()

            def loop_b(i, _):
                s = pl.ds(i * L, L)
                w = offv[s]
                o0 = w & 255
                o1 = (w >> 8) & 255
                o2 = (w >> 16) & 255
                i0 = (hxv[s] + o0 * oscale) & (T - 1)
                i1 = (hyv[s] + o1 * oscale) & (T - 1)
                i2 = (hzv[s] + o2 * oscale) & (T - 1)
                idxv[s] = (i0 * T + i1) * T + i2
                return 0

            lax.fori_loop(0, CH // L, loop_b, 0)

            cps = [
                pltpu.async_copy(
                    sp_h.at[idxv.at[pl.ds(w * GW, GW)]],
                    stv.at[pl.ds(w * GW, GW)], sem)
                for w in range(CH // GW)
            ]
            for cp in cps:
                cp.wait()

            def loop_c(i, cnt):
                s = pl.ds(i * L, L)
                m = stv[s] == cbv[s]
                plsc.store_compressed(hitrv.at[pl.ds(cnt, L)], idxv[s], mask=m)
                dvec = (base + i * L) + iota
                plsc.store_compressed(hitdv.at[pl.ds(cnt, L)], dvec, mask=m)
                return cnt + jnp.sum(m.astype(jnp.int32))

            cnt = lax.fori_loop(0, CH // L, loop_c, 0)

            pltpu.sync_copy(zerov, out_h.at[pl.ds(base * 16, CH * 16)])

            ng = (cnt + (L - 1)) // L

            def hit_group(g, _):
                rv = hitrv[pl.ds(g * L, L)]
                dv = hitdv[pl.ds(g * L, L)]
                live = cnt - g * L
                for l in range(L):
                    @pl.when(l < live)
                    def _():
                        r_l = jnp.sum(jnp.where(iota == l, rv, 0))
                        pltpu.async_copy(
                            tbl_h.at[r_l * C + iota], stagev.at[l], sem2)
                nlive = jnp.minimum(live, L)

                def drain2(j, _):
                    pltpu.make_async_copy(
                        tbl_h.at[pl.ds(0, L)], dumv, sem2).wait()
                    return 0

                lax.fori_loop(0, nlive, drain2, 0)
                for l in range(L):
                    @pl.when(l < live)
                    def _():
                        d_l = jnp.sum(jnp.where(iota == l, dv, 0))
                        pltpu.async_copy(
                            stagev.at[l], out_h.at[d_l * C + iota], sem3)

                def drain3(j, _):
                    pltpu.make_async_copy(
                        tbl_h.at[pl.ds(0, L)], dumv, sem3).wait()
                    return 0

                lax.fori_loop(0, nlive, drain3, 0)
                return 0

            lax.fori_loop(0, ng, hit_group, 0)
            return 0

        lax.fori_loop(0, R, round_body, 0)

    return kern


def kernel(points, hash_table, offset_table, sparsity_encoding, m0, m1):
    T = hash_table.shape[0]
    O = offset_table.shape[0]
    C = hash_table.shape[-1]
    N = points.shape[0]
    oscale = int(np.ceil(T / 255.0))

    pts = jnp.pad(points, ((0, NPAD - N), (0, 0)))
    ptsT = pts.T
    px, py, pz = ptsT[0], ptsT[1], ptsT[2]

    tbl = hash_table.reshape(T * T * T * C)

    op = offset_table.reshape(O * O * O, 3)
    offp = op[:, 0] + op[:, 1] * 256 + op[:, 2] * 65536

    sp = sparsity_encoding.reshape(T * T * T)

    pf = jnp.arange(T, dtype=jnp.float32)
    ttab = pf * lax.rsqrt(pf + jnp.float32(float(1) * C1))

    mm = jnp.zeros((8, 16), jnp.float32)
    mm = mm.at[0:3, :].set(jnp.broadcast_to(m0[:, None], (3, 16)))
    mm = mm.at[3:6, :].set(jnp.broadcast_to(m1[:, None], (3, 16)))

    out = _sc_hash_lookup(T, O, C, oscale)(px, py, pz, tbl, offp, sp, ttab, mm)
    return out[:N * C].reshape(N, C)
